# Initial kernel scaffold; baseline (speedup 1.0000x reference)
import jax
import jax.numpy as jnp
from jax.experimental import pallas as pl


def kernel(node_attr, edge_index, edge_attr, edge_sh, W1, b1, W2, b2):
    raise NotImplementedError("write your pallas kernel here")


# speedup vs baseline: 2.7042x; 2.7042x over previous
import functools
import numpy as np
import jax
import jax.numpy as jnp
from jax import lax
from jax.experimental import pallas as pl
from jax.experimental.pallas import tpu as pltpu
from jax.experimental.pallas import tpu_sc as plsc

N = 10000
E = 160000
MUL_IN = 16
SH_DIM = 4
MUL0_OUT = 16
MUL1_OUT = 8
D_EDGE = 16
HID = 128
W_NUMEL = MUL_IN * MUL0_OUT + MUL_IN * MUL1_OUT
OUT_DIM = MUL0_OUT + 3 * MUL1_OUT
PAD_DIM = 48
NORM = 1.0 / np.sqrt(MUL_IN)

NC = 2
NS = 16
NW = NC * NS
EPW = E // NW
CHUNK = 40
NCHUNK = EPW // CHUNK
N_PAD = 10240
NPT = N_PAD // NS


def _build_consts():
    q = np.zeros((MUL_IN, W_NUMEL), np.float32)
    for i in range(MUL_IN):
        q[i, i * MUL0_OUT:(i + 1) * MUL0_OUT] = 1.0
        base = MUL_IN * MUL0_OUT
        q[i, base + i * MUL1_OUT: base + (i + 1) * MUL1_OUT] = 1.0
    m = np.zeros((W_NUMEL, PAD_DIM), np.float32)
    for i in range(MUL_IN):
        for o in range(MUL0_OUT):
            m[i * MUL0_OUT + o, o] = NORM
        base = MUL_IN * MUL0_OUT
        for o in range(MUL1_OUT):
            for c in range(3):
                m[base + i * MUL1_OUT + o, MUL0_OUT + o * 3 + c] = NORM
    shm = np.zeros((SH_DIM, PAD_DIM), np.float32)
    shm[0, :MUL0_OUT] = 1.0
    for o in range(MUL1_OUT):
        for c in range(3):
            shm[1 + c, MUL0_OUT + o * 3 + c] = 1.0
    e40 = np.zeros((PAD_DIM,), np.float32)
    e40[OUT_DIM] = 1.0
    return jnp.asarray(q), jnp.asarray(m), jnp.asarray(shm), jnp.asarray(e40)


def _gather_body(nodes_hbm, dst_hbm, out_hbm, idx_v, rows_v, sem):
    wid = lax.axis_index("s") * NC + lax.axis_index("c")
    pltpu.sync_copy(dst_hbm.at[wid], idx_v)

    def chunk(j, _):
        pltpu.async_copy(nodes_hbm.at[idx_v.at[j]], rows_v, sem).wait()
        pltpu.sync_copy(rows_v, out_hbm.at[pl.ds(wid * EPW + j * CHUNK, CHUNK)])
        return 0

    lax.fori_loop(0, NCHUNK, chunk, 0)


@jax.jit
def _sc_gather(node_attr, dst_r):
    mesh = plsc.VectorSubcoreMesh(core_axis_name="c", subcore_axis_name="s")
    return pl.kernel(
        _gather_body,
        out_type=jax.ShapeDtypeStruct((E, MUL_IN), jnp.float32),
        mesh=mesh,
        scratch_types=[
            pltpu.VMEM((NCHUNK, CHUNK), jnp.int32),
            pltpu.VMEM((CHUNK, MUL_IN), jnp.float32),
            pltpu.SemaphoreType.DMA,
        ],
        compiler_params=pltpu.CompilerParams(use_tc_tiling_on_sc=False),
    )(node_attr, dst_r)


TC_B = 1280


def _dense_body(ea_ref, xd_ref, sh_ref, w1_ref, b1_ref, w2_ref, b2_ref,
                q_ref, m_ref, shm_ref, e40_ref, out_ref):
    f32 = jnp.float32
    h = jnp.maximum(
        jnp.dot(ea_ref[...], w1_ref[...], preferred_element_type=f32)
        + b1_ref[...], 0.0)
    w = jnp.dot(h, w2_ref[...], preferred_element_type=f32) + b2_ref[...]
    u = w * jnp.dot(xd_ref[...], q_ref[...], preferred_element_type=f32)
    tp = (jnp.dot(u, m_ref[...], preferred_element_type=f32)
          * jnp.dot(sh_ref[...], shm_ref[...], preferred_element_type=f32))
    out_ref[...] = tp + e40_ref[...]


@jax.jit
def _tc_dense(edge_attr, xd, edge_sh, W1, b1, W2, b2, Q, M, SHM, E40):
    grid = (E // TC_B,)
    full = lambda r, c: pl.BlockSpec((r, c), lambda i: (0, 0))
    return pl.pallas_call(
        _dense_body,
        grid=grid,
        in_specs=[
            pl.BlockSpec((TC_B, D_EDGE), lambda i: (i, 0)),
            pl.BlockSpec((TC_B, MUL_IN), lambda i: (i, 0)),
            pl.BlockSpec((TC_B, SH_DIM), lambda i: (i, 0)),
            full(D_EDGE, HID),
            full(1, HID),
            full(HID, W_NUMEL),
            full(1, W_NUMEL),
            full(MUL_IN, W_NUMEL),
            full(W_NUMEL, PAD_DIM),
            full(SH_DIM, PAD_DIM),
            full(1, PAD_DIM),
        ],
        out_specs=pl.BlockSpec((TC_B, PAD_DIM), lambda i: (i, 0)),
        out_shape=jax.ShapeDtypeStruct((E, PAD_DIM), jnp.float32),
        compiler_params=pltpu.CompilerParams(
            dimension_semantics=("arbitrary",)),
    )(edge_attr, xd, edge_sh, W1, b1.reshape(1, HID), W2,
      b2.reshape(1, W_NUMEL), Q, M, SHM, E40.reshape(1, PAD_DIM))


def _scatter_body(tp_hbm, src_hbm, zeros_hbm, out_hbm, idx_v, rows_v, acc, sem):
    c = lax.axis_index("c")
    s = lax.axis_index("s")
    wid = s * NC + c
    pltpu.sync_copy(zeros_hbm.at[pl.ds(s * NPT, NPT)],
                    acc.at[pl.ds(s * NPT, NPT)])
    plsc.subcore_barrier()
    pltpu.sync_copy(src_hbm.at[wid], idx_v)

    def chunk(j, _):
        pltpu.sync_copy(tp_hbm.at[pl.ds(wid * EPW + j * CHUNK, CHUNK)], rows_v)
        pltpu.sync_copy(rows_v, acc.at[idx_v.at[j]], add=True)
        return 0

    lax.fori_loop(0, NCHUNK, chunk, 0)
    plsc.subcore_barrier()
    pltpu.sync_copy(acc.at[pl.ds(s * NPT, NPT)],
                    out_hbm.at[c, pl.ds(s * NPT, NPT)])


@jax.jit
def _sc_scatter(tp, src_r, zeros):
    mesh = plsc.VectorSubcoreMesh(core_axis_name="c", subcore_axis_name="s")
    return pl.kernel(
        _scatter_body,
        out_type=jax.ShapeDtypeStruct((NC, N_PAD, PAD_DIM), jnp.float32),
        mesh=mesh,
        scratch_types=[
            pltpu.VMEM((NCHUNK, CHUNK), jnp.int32),
            pltpu.VMEM((CHUNK, PAD_DIM), jnp.float32),
            pltpu.VMEM_SHARED((N_PAD, PAD_DIM), jnp.float32),
            pltpu.SemaphoreType.DMA,
        ],
        compiler_params=pltpu.CompilerParams(use_tc_tiling_on_sc=False),
    )(tp, src_r, zeros)


CB_B = 2000


def _combine_body(p_ref, out_ref):
    ssum = p_ref[0] + p_ref[1]
    col = lax.broadcasted_iota(jnp.int32, (CB_B, PAD_DIM), 1)
    cnt = jnp.max(jnp.where(col == OUT_DIM, ssum, 0.0), axis=1, keepdims=True)
    out_ref[...] = ssum / jnp.maximum(cnt, 1.0)


@jax.jit
def _tc_combine(partial):
    return pl.pallas_call(
        _combine_body,
        grid=(N // CB_B,),
        in_specs=[pl.BlockSpec((NC, CB_B, PAD_DIM), lambda i: (0, i, 0))],
        out_specs=pl.BlockSpec((CB_B, PAD_DIM), lambda i: (i, 0)),
        out_shape=jax.ShapeDtypeStruct((N, PAD_DIM), jnp.float32),
        compiler_params=pltpu.CompilerParams(
            dimension_semantics=("arbitrary",)),
    )(partial)


def kernel(node_attr, edge_index, edge_attr, edge_sh, W1, b1, W2, b2):
    Q, M, SHM, E40 = _build_consts()
    src_r = edge_index[0].reshape(NW, NCHUNK, CHUNK)
    dst_r = edge_index[1].reshape(NW, NCHUNK, CHUNK)
    zeros = jnp.zeros((N_PAD, PAD_DIM), jnp.float32)
    xd = _sc_gather(node_attr, dst_r)
    tp = _tc_dense(edge_attr, xd, edge_sh, W1, b1, W2, b2, Q, M, SHM, E40)
    partial = _sc_scatter(tp, src_r, zeros)
    out48 = _tc_combine(partial)
    return out48[:, :OUT_DIM]


# speedup vs baseline: 3.2882x; 1.2160x over previous
import functools
import numpy as np
import jax
import jax.numpy as jnp
from jax import lax
from jax.experimental import pallas as pl
from jax.experimental.pallas import tpu as pltpu
from jax.experimental.pallas import tpu_sc as plsc

N = 10000
E = 160000
MUL_IN = 16
SH_DIM = 4
MUL0_OUT = 16
MUL1_OUT = 8
D_EDGE = 16
HID = 128
W_NUMEL = MUL_IN * MUL0_OUT + MUL_IN * MUL1_OUT
OUT_DIM = MUL0_OUT + 3 * MUL1_OUT
PAD_DIM = 48
NORM = 1.0 / np.sqrt(MUL_IN)

NC = 2
NS = 16
NW = NC * NS
EPW = E // NW
CHUNK = 40
NCHUNK = EPW // CHUNK
N_PAD = 10240
NPT = N_PAD // NS


def _build_consts():
    q = np.zeros((MUL_IN, W_NUMEL), np.float32)
    for i in range(MUL_IN):
        q[i, i * MUL0_OUT:(i + 1) * MUL0_OUT] = 1.0
        base = MUL_IN * MUL0_OUT
        q[i, base + i * MUL1_OUT: base + (i + 1) * MUL1_OUT] = 1.0
    m = np.zeros((W_NUMEL, PAD_DIM), np.float32)
    for i in range(MUL_IN):
        for o in range(MUL0_OUT):
            m[i * MUL0_OUT + o, o] = NORM
        base = MUL_IN * MUL0_OUT
        for o in range(MUL1_OUT):
            for c in range(3):
                m[base + i * MUL1_OUT + o, MUL0_OUT + o * 3 + c] = NORM
    shm = np.zeros((SH_DIM, PAD_DIM), np.float32)
    shm[0, :MUL0_OUT] = 1.0
    for o in range(MUL1_OUT):
        for c in range(3):
            shm[1 + c, MUL0_OUT + o * 3 + c] = 1.0
    e40 = np.zeros((PAD_DIM,), np.float32)
    e40[OUT_DIM] = 1.0
    return jnp.asarray(q), jnp.asarray(m), jnp.asarray(shm), jnp.asarray(e40)


def _gather_body(nodes_hbm, dst_hbm, out_hbm, idx_v, rows_v, sem):
    wid = lax.axis_index("s") * NC + lax.axis_index("c")
    base = wid * EPW
    pltpu.sync_copy(dst_hbm.at[pl.ds(base, EPW)], idx_v)
    pltpu.async_copy(nodes_hbm.at[idx_v], rows_v, sem).wait()
    pltpu.sync_copy(rows_v, out_hbm.at[pl.ds(base, EPW)])


@jax.jit
def _sc_gather(node_attr, dst):
    mesh = plsc.VectorSubcoreMesh(core_axis_name="c", subcore_axis_name="s")
    return pl.kernel(
        _gather_body,
        out_type=jax.ShapeDtypeStruct((E, MUL_IN), jnp.float32),
        mesh=mesh,
        scratch_types=[
            pltpu.VMEM((EPW,), jnp.int32),
            pltpu.VMEM((EPW, MUL_IN), jnp.float32),
            pltpu.SemaphoreType.DMA,
        ],
        compiler_params=pltpu.CompilerParams(use_tc_tiling_on_sc=False),
    )(node_attr, dst)


TC_B = 1280


def _dense_body(ea_ref, xd_ref, sh_ref, w1_ref, b1_ref, w2_ref, b2_ref,
                q_ref, m_ref, shm_ref, e40_ref, out_ref):
    f32 = jnp.float32
    h = jnp.maximum(
        jnp.dot(ea_ref[...], w1_ref[...], preferred_element_type=f32)
        + b1_ref[...], 0.0)
    w = jnp.dot(h, w2_ref[...], preferred_element_type=f32) + b2_ref[...]
    u = w * jnp.dot(xd_ref[...], q_ref[...], preferred_element_type=f32)
    tp = (jnp.dot(u, m_ref[...], preferred_element_type=f32)
          * jnp.dot(sh_ref[...], shm_ref[...], preferred_element_type=f32))
    out_ref[...] = tp + e40_ref[...]


@jax.jit
def _tc_dense(edge_attr, xd, edge_sh, W1, b1, W2, b2, Q, M, SHM, E40):
    grid = (E // TC_B,)
    full = lambda r, c: pl.BlockSpec((r, c), lambda i: (0, 0))
    return pl.pallas_call(
        _dense_body,
        grid=grid,
        in_specs=[
            pl.BlockSpec((TC_B, D_EDGE), lambda i: (i, 0)),
            pl.BlockSpec((TC_B, MUL_IN), lambda i: (i, 0)),
            pl.BlockSpec((TC_B, SH_DIM), lambda i: (i, 0)),
            full(D_EDGE, HID),
            full(1, HID),
            full(HID, W_NUMEL),
            full(1, W_NUMEL),
            full(MUL_IN, W_NUMEL),
            full(W_NUMEL, PAD_DIM),
            full(SH_DIM, PAD_DIM),
            full(1, PAD_DIM),
        ],
        out_specs=pl.BlockSpec((TC_B, PAD_DIM), lambda i: (i, 0)),
        out_shape=jax.ShapeDtypeStruct((E, PAD_DIM), jnp.float32),
        compiler_params=pltpu.CompilerParams(
            dimension_semantics=("arbitrary",)),
    )(edge_attr, xd, edge_sh, W1, b1.reshape(1, HID), W2,
      b2.reshape(1, W_NUMEL), Q, M, SHM, E40.reshape(1, PAD_DIM))


RND = 8
RROWS = EPW // RND


def _scatter_body(tp_hbm, src_hbm, zeros_hbm, out_hbm, idx_v, rows_a, rows_b,
                  acc, lsem, ssem):
    c = lax.axis_index("c")
    s = lax.axis_index("s")
    wid = s * NC + c
    base = wid * EPW
    pltpu.sync_copy(zeros_hbm.at[pl.ds(s * NPT, NPT)],
                    acc.at[pl.ds(s * NPT, NPT)])
    pltpu.sync_copy(src_hbm.at[wid], idx_v)
    plsc.subcore_barrier()
    bufs = (rows_a, rows_b)
    load = pltpu.async_copy(tp_hbm.at[pl.ds(base, RROWS)], rows_a, lsem)
    for r in range(RND):
        cur = bufs[r % 2]
        load.wait()
        if r + 1 < RND:
            load = pltpu.async_copy(
                tp_hbm.at[pl.ds(base + (r + 1) * RROWS, RROWS)],
                bufs[(r + 1) % 2], lsem)
        pltpu.async_copy(cur, acc.at[idx_v.at[r]], ssem, add=True).wait()
    plsc.subcore_barrier()
    pltpu.sync_copy(acc.at[pl.ds(s * NPT, NPT)],
                    out_hbm.at[c, pl.ds(s * NPT, NPT)])


@jax.jit
def _sc_scatter(tp, src_r, zeros):
    mesh = plsc.VectorSubcoreMesh(core_axis_name="c", subcore_axis_name="s")
    return pl.kernel(
        _scatter_body,
        out_type=jax.ShapeDtypeStruct((NC, N_PAD, PAD_DIM), jnp.float32),
        mesh=mesh,
        scratch_types=[
            pltpu.VMEM((RND, RROWS), jnp.int32),
            pltpu.VMEM((RROWS, PAD_DIM), jnp.float32),
            pltpu.VMEM((RROWS, PAD_DIM), jnp.float32),
            pltpu.VMEM_SHARED((N_PAD, PAD_DIM), jnp.float32),
            pltpu.SemaphoreType.DMA,
            pltpu.SemaphoreType.DMA,
        ],
        compiler_params=pltpu.CompilerParams(use_tc_tiling_on_sc=False),
    )(tp, src_r, zeros)


CB_B = 2000


def _combine_body(p_ref, out_ref):
    ssum = p_ref[0] + p_ref[1]
    col = lax.broadcasted_iota(jnp.int32, (CB_B, PAD_DIM), 1)
    cnt = jnp.max(jnp.where(col == OUT_DIM, ssum, 0.0), axis=1, keepdims=True)
    out_ref[...] = ssum / jnp.maximum(cnt, 1.0)


@jax.jit
def _tc_combine(partial):
    return pl.pallas_call(
        _combine_body,
        grid=(N // CB_B,),
        in_specs=[pl.BlockSpec((NC, CB_B, PAD_DIM), lambda i: (0, i, 0))],
        out_specs=pl.BlockSpec((CB_B, PAD_DIM), lambda i: (i, 0)),
        out_shape=jax.ShapeDtypeStruct((N, PAD_DIM), jnp.float32),
        compiler_params=pltpu.CompilerParams(
            dimension_semantics=("arbitrary",)),
    )(partial)


def kernel(node_attr, edge_index, edge_attr, edge_sh, W1, b1, W2, b2):
    Q, M, SHM, E40 = _build_consts()
    src_r = edge_index[0].reshape(NW, RND, RROWS)
    zeros = jnp.zeros((N_PAD, PAD_DIM), jnp.float32)
    xd = _sc_gather(node_attr, edge_index[1])
    tp = _tc_dense(edge_attr, xd, edge_sh, W1, b1, W2, b2, Q, M, SHM, E40)
    partial = _sc_scatter(tp, src_r, zeros)
    out48 = _tc_combine(partial)
    return out48[:, :OUT_DIM]


# speedup vs baseline: 4.0031x; 1.2174x over previous
import numpy as np
import jax
import jax.numpy as jnp
from jax import lax
from jax.experimental import pallas as pl
from jax.experimental.pallas import tpu as pltpu
from jax.experimental.pallas import tpu_sc as plsc

N = 10000
E = 160000
EH = E // 2
MUL_IN = 16
SH_DIM = 4
MUL0_OUT = 16
MUL1_OUT = 8
D_EDGE = 16
HID = 128
W_NUMEL = MUL_IN * MUL0_OUT + MUL_IN * MUL1_OUT
OUT_DIM = MUL0_OUT + 3 * MUL1_OUT
PAD_DIM = 64
NORM = 1.0 / np.sqrt(MUL_IN)

NC = 2
NS = 16
NW = NC * NS
EPW = E // NW
N_PAD = 10240
NPT = N_PAD // NS
RND = 8
RROWS = EPW // RND


def _build_consts():
    q = np.zeros((MUL_IN, W_NUMEL), np.float32)
    for i in range(MUL_IN):
        q[i, i * MUL0_OUT:(i + 1) * MUL0_OUT] = 1.0
        base = MUL_IN * MUL0_OUT
        q[i, base + i * MUL1_OUT: base + (i + 1) * MUL1_OUT] = 1.0
    m = np.zeros((W_NUMEL, PAD_DIM), np.float32)
    for i in range(MUL_IN):
        for o in range(MUL0_OUT):
            m[i * MUL0_OUT + o, o] = NORM
        base = MUL_IN * MUL0_OUT
        for o in range(MUL1_OUT):
            for c in range(3):
                m[base + i * MUL1_OUT + o, MUL0_OUT + o * 3 + c] = NORM
    shm = np.zeros((SH_DIM, PAD_DIM), np.float32)
    shm[0, :MUL0_OUT] = 1.0
    for o in range(MUL1_OUT):
        for c in range(3):
            shm[1 + c, MUL0_OUT + o * 3 + c] = 1.0
    e40 = np.zeros((PAD_DIM,), np.float32)
    e40[OUT_DIM] = 1.0
    return jnp.asarray(q), jnp.asarray(m), jnp.asarray(shm), jnp.asarray(e40)


G_RND = 20
G_ROWS = EPW // G_RND


def _gather_body(nodes_hbm, dst_hbm, out_hbm, idx_v, rows_a, rows_b,
                 gsem, wsem):
    wid = lax.axis_index("s") * NC + lax.axis_index("c")
    base = wid * EPW
    pltpu.sync_copy(dst_hbm.at[wid], idx_v)
    bufs = (rows_a, rows_b)
    g = pltpu.async_copy(nodes_hbm.at[idx_v.at[0]], rows_a, gsem)
    w = None
    for r in range(G_RND):
        cur = bufs[r % 2]
        g.wait()
        if w is not None:
            w.wait()
        if r + 1 < G_RND:
            g = pltpu.async_copy(nodes_hbm.at[idx_v.at[r + 1]],
                                 bufs[(r + 1) % 2], gsem)
        w = pltpu.async_copy(cur, out_hbm.at[pl.ds(base + r * G_ROWS,
                                                   G_ROWS)], wsem)
    w.wait()


@jax.jit
def _sc_gather(node_pad, dst_r):
    mesh = plsc.VectorSubcoreMesh(core_axis_name="c", subcore_axis_name="s")
    return pl.kernel(
        _gather_body,
        out_type=jax.ShapeDtypeStruct((E, 128), jnp.float32),
        mesh=mesh,
        scratch_types=[
            pltpu.VMEM((G_RND, G_ROWS), jnp.int32),
            pltpu.VMEM((G_ROWS, 128), jnp.float32),
            pltpu.VMEM((G_ROWS, 128), jnp.float32),
            pltpu.SemaphoreType.DMA,
            pltpu.SemaphoreType.DMA,
        ],
        compiler_params=pltpu.CompilerParams(use_tc_tiling_on_sc=False),
    )(node_pad, dst_r)


TC_B = 1600
GRID = EH // TC_B
def _half(ea, xd128, sh, w1, b1, w2, b2, q, m, shm):
    f32 = jnp.float32
    xd = xd128[:, :MUL_IN]
    h = jnp.maximum(jnp.dot(ea, w1, preferred_element_type=f32) + b1, 0.0)
    w = jnp.dot(h, w2, preferred_element_type=f32) + b2
    u = w * jnp.dot(xd, q, preferred_element_type=f32)
    return (jnp.dot(u, m, preferred_element_type=f32)
            * jnp.dot(sh, shm, preferred_element_type=f32))


def _dense_body(ea_a, ea_b, xd_a, xd_b, sh_a, sh_b, w1_ref, b1_ref, w2_ref,
                b2_ref, q_ref, m_ref, shm_ref, e40_ref, out_ref):
    w1, b1 = w1_ref[...], b1_ref[...]
    w2, b2 = w2_ref[...], b2_ref[...]
    q, m, shm, e40 = q_ref[...], m_ref[...], shm_ref[...], e40_ref[...]
    tpa = _half(ea_a[...], xd_a[...], sh_a[...], w1, b1, w2, b2, q, m, shm)
    tpb = _half(ea_b[...], xd_b[...], sh_b[...], w1, b1, w2, b2, q, m, shm)
    out_ref[...] = jnp.concatenate([tpa + e40, tpb + e40], axis=1)


@jax.jit
def _tc_dense(ea, xd128, sh, W1, b1, W2, b2, Q, M, SHM, E40):
    full = lambda r, c: pl.BlockSpec((r, c), lambda i: (0, 0))
    return pl.pallas_call(
        _dense_body,
        grid=(GRID,),
        in_specs=[
            pl.BlockSpec((TC_B, D_EDGE), lambda i: (i, 0)),
            pl.BlockSpec((TC_B, D_EDGE), lambda i: (i + GRID, 0)),
            pl.BlockSpec((TC_B, 128), lambda i: (i, 0)),
            pl.BlockSpec((TC_B, 128), lambda i: (i + GRID, 0)),
            pl.BlockSpec((TC_B, SH_DIM), lambda i: (i, 0)),
            pl.BlockSpec((TC_B, SH_DIM), lambda i: (i + GRID, 0)),
            full(D_EDGE, HID),
            full(1, HID),
            full(HID, W_NUMEL),
            full(1, W_NUMEL),
            full(MUL_IN, W_NUMEL),
            full(W_NUMEL, PAD_DIM),
            full(SH_DIM, PAD_DIM),
            full(1, PAD_DIM),
        ],
        out_specs=pl.BlockSpec((TC_B, 128), lambda i: (i, 0)),
        out_shape=jax.ShapeDtypeStruct((EH, 128), jnp.float32),
        compiler_params=pltpu.CompilerParams(
            dimension_semantics=("arbitrary",)),
    )(ea, ea, xd128, xd128, sh, sh, W1, b1.reshape(1, HID), W2,
      b2.reshape(1, W_NUMEL), Q, M, SHM, E40.reshape(1, PAD_DIM))


def _scatter_body(tp_hbm, src_hbm, zeros_hbm, out_hbm, idx_v, rows_a, rows_b,
                  acc, lsem, ssem):
    c = lax.axis_index("c")
    s = lax.axis_index("s")
    wid = s * NC + c
    base = wid * EPW
    pltpu.sync_copy(zeros_hbm.at[pl.ds(s * NPT, NPT)],
                    acc.at[pl.ds(s * NPT, NPT)])
    pltpu.sync_copy(src_hbm.at[wid], idx_v)
    plsc.subcore_barrier()
    bufs = (rows_a, rows_b)
    load = pltpu.async_copy(tp_hbm.at[pl.ds(base, RROWS)], rows_a, lsem)
    for r in range(RND):
        cur = bufs[r % 2]
        load.wait()
        if r + 1 < RND:
            load = pltpu.async_copy(
                tp_hbm.at[pl.ds(base + (r + 1) * RROWS, RROWS)],
                bufs[(r + 1) % 2], lsem)
        pltpu.async_copy(cur, acc.at[idx_v.at[r]], ssem, add=True).wait()
    plsc.subcore_barrier()
    pltpu.sync_copy(acc.at[pl.ds(s * NPT, NPT)],
                    out_hbm.at[c, pl.ds(s * NPT, NPT)])


@jax.jit
def _sc_scatter(tp, src_r, zeros):
    mesh = plsc.VectorSubcoreMesh(core_axis_name="c", subcore_axis_name="s")
    return pl.kernel(
        _scatter_body,
        out_type=jax.ShapeDtypeStruct((NC, N_PAD, PAD_DIM), jnp.float32),
        mesh=mesh,
        scratch_types=[
            pltpu.VMEM((RND, RROWS), jnp.int32),
            pltpu.VMEM((RROWS, PAD_DIM), jnp.float32),
            pltpu.VMEM((RROWS, PAD_DIM), jnp.float32),
            pltpu.VMEM_SHARED((N_PAD, PAD_DIM), jnp.float32),
            pltpu.SemaphoreType.DMA,
            pltpu.SemaphoreType.DMA,
        ],
        compiler_params=pltpu.CompilerParams(use_tc_tiling_on_sc=False),
    )(tp, src_r, zeros)


CB_B = N_PAD // 8
CNT_A = OUT_DIM
CNT_B = PAD_DIM + OUT_DIM


def _combine_body(p_ref, out_ref):
    s = p_ref[0] + p_ref[1]
    col = lax.broadcasted_iota(jnp.int32, (CB_B, 128), 1)
    cnt_a = jnp.max(jnp.where(col == CNT_A, s, 0.0), axis=1, keepdims=True)
    cnt_b = jnp.max(jnp.where(col == CNT_B, s, 0.0), axis=1, keepdims=True)
    cnt = jnp.where(col < PAD_DIM, jnp.maximum(cnt_a, 1.0),
                    jnp.maximum(cnt_b, 1.0))
    out_ref[...] = s / cnt


@jax.jit
def _tc_combine(partial):
    return pl.pallas_call(
        _combine_body,
        grid=(N_PAD // 2 // CB_B,),
        in_specs=[pl.BlockSpec((NC, CB_B, 128), lambda i: (0, i, 0))],
        out_specs=pl.BlockSpec((CB_B, 128), lambda i: (i, 0)),
        out_shape=jax.ShapeDtypeStruct((N_PAD // 2, 128), jnp.float32),
        compiler_params=pltpu.CompilerParams(
            dimension_semantics=("arbitrary",)),
    )(partial)


def kernel(node_attr, edge_index, edge_attr, edge_sh, W1, b1, W2, b2):
    Q, M, SHM, E40 = _build_consts()
    src = edge_index[0]
    src_perm = jnp.stack([src[:EH], src[EH:]], axis=1).reshape(E)
    src_r = src_perm.reshape(NW, RND, RROWS)
    zeros = jnp.zeros((N_PAD, PAD_DIM), jnp.float32)
    node_pad = jnp.pad(node_attr, ((0, 0), (0, 128 - MUL_IN)))
    xd128 = _sc_gather(node_pad, edge_index[1].reshape(NW, G_RND, G_ROWS))
    tp2 = _tc_dense(edge_attr, xd128, edge_sh, W1, b1, W2, b2, Q, M, SHM, E40)
    tp = tp2.reshape(E, PAD_DIM)
    partial = _sc_scatter(tp, src_r, zeros)
    out2 = _tc_combine(partial.reshape(NC, N_PAD // 2, 128))
    return out2.reshape(N_PAD, PAD_DIM)[:N, :OUT_DIM]


# speedup vs baseline: 5.0588x; 1.2637x over previous
import numpy as np
import jax
import jax.numpy as jnp
from jax import lax
from jax.experimental import pallas as pl
from jax.experimental.pallas import tpu as pltpu
from jax.experimental.pallas import tpu_sc as plsc

N = 10000
E = 160000
EH = E // 2
MUL_IN = 16
SH_DIM = 4
MUL0_OUT = 16
MUL1_OUT = 8
D_EDGE = 16
HID = 128
W_NUMEL = MUL_IN * MUL0_OUT + MUL_IN * MUL1_OUT
OUT_DIM = MUL0_OUT + 3 * MUL1_OUT
PAD_DIM = 64
NORM = 1.0 / np.sqrt(MUL_IN)

NC = 2
NS = 16
NW = NC * NS
EPW = E // NW
N_PAD = 10240
NPT = N_PAD // NS
RND = 8
RROWS = EPW // RND


def _build_consts():
    q = np.zeros((MUL_IN, W_NUMEL), np.float32)
    for i in range(MUL_IN):
        q[i, i * MUL0_OUT:(i + 1) * MUL0_OUT] = 1.0
        base = MUL_IN * MUL0_OUT
        q[i, base + i * MUL1_OUT: base + (i + 1) * MUL1_OUT] = 1.0
    m = np.zeros((W_NUMEL, PAD_DIM), np.float32)
    for i in range(MUL_IN):
        for o in range(MUL0_OUT):
            m[i * MUL0_OUT + o, o] = NORM
        base = MUL_IN * MUL0_OUT
        for o in range(MUL1_OUT):
            for c in range(3):
                m[base + i * MUL1_OUT + o, MUL0_OUT + o * 3 + c] = NORM
    shm = np.zeros((SH_DIM, PAD_DIM), np.float32)
    shm[0, :MUL0_OUT] = 1.0
    for o in range(MUL1_OUT):
        for c in range(3):
            shm[1 + c, MUL0_OUT + o * 3 + c] = 1.0
    e40 = np.zeros((PAD_DIM,), np.float32)
    e40[OUT_DIM] = 1.0
    return jnp.asarray(q), jnp.asarray(m), jnp.asarray(shm), jnp.asarray(e40)


def _gather_body(nodes_hbm, dst_hbm, out_hbm, idx_v, rows_v, sem):
    wid = lax.axis_index("s") * NC + lax.axis_index("c")
    base = wid * EPW
    pltpu.sync_copy(dst_hbm.at[pl.ds(base, EPW)], idx_v)
    pltpu.async_copy(nodes_hbm.at[idx_v], rows_v, sem).wait()
    pltpu.sync_copy(rows_v, out_hbm.at[pl.ds(base, EPW)])


@jax.jit
def _sc_gather(node_attr, dst):
    mesh = plsc.VectorSubcoreMesh(core_axis_name="c", subcore_axis_name="s")
    return pl.kernel(
        _gather_body,
        out_type=jax.ShapeDtypeStruct((E, MUL_IN), jnp.float32),
        mesh=mesh,
        scratch_types=[
            pltpu.VMEM((EPW,), jnp.int32),
            pltpu.VMEM((EPW, MUL_IN), jnp.float32),
            pltpu.SemaphoreType.DMA,
        ],
        compiler_params=pltpu.CompilerParams(use_tc_tiling_on_sc=False),
    )(node_attr, dst)


TC_B = 3200
GRID = EH // TC_B
def _half(ea_t, xd_t, sh_t, w1, b1, w2, b2, q, m, shm):
    f32 = jnp.float32
    ea = ea_t.T
    xd = xd_t
    sh = sh_t.T
    h = jnp.maximum(jnp.dot(ea, w1, preferred_element_type=f32) + b1, 0.0)
    w = jnp.dot(h, w2, preferred_element_type=f32) + b2
    u = w * jnp.dot(xd, q, preferred_element_type=f32)
    return (jnp.dot(u, m, preferred_element_type=f32)
            * jnp.dot(sh, shm, preferred_element_type=f32))


def _dense_body(ea_a, ea_b, xd_a, xd_b, sh_a, sh_b, w1_ref, b1_ref, w2_ref,
                b2_ref, q_ref, m_ref, shm_ref, e40_ref, out_ref):
    w1, b1 = w1_ref[...], b1_ref[...]
    w2, b2 = w2_ref[...], b2_ref[...]
    q, m, shm, e40 = q_ref[...], m_ref[...], shm_ref[...], e40_ref[...]
    tpa = _half(ea_a[...], xd_a[...], sh_a[...], w1, b1, w2, b2, q, m, shm)
    tpb = _half(ea_b[...], xd_b[...], sh_b[...], w1, b1, w2, b2, q, m, shm)
    out_ref[...] = jnp.concatenate([tpa + e40, tpb + e40], axis=1)


@jax.jit
def _tc_dense(eaT, xdT, shT, W1, b1, W2, b2, Q, M, SHM, E40):
    full = lambda r, c: pl.BlockSpec((r, c), lambda i: (0, 0))
    return pl.pallas_call(
        _dense_body,
        grid=(GRID,),
        in_specs=[
            pl.BlockSpec((D_EDGE, TC_B), lambda i: (0, i)),
            pl.BlockSpec((D_EDGE, TC_B), lambda i: (0, i + GRID)),
            pl.BlockSpec((TC_B, MUL_IN), lambda i: (i, 0)),
            pl.BlockSpec((TC_B, MUL_IN), lambda i: (i + GRID, 0)),
            pl.BlockSpec((SH_DIM, TC_B), lambda i: (0, i)),
            pl.BlockSpec((SH_DIM, TC_B), lambda i: (0, i + GRID)),
            full(D_EDGE, HID),
            full(1, HID),
            full(HID, W_NUMEL),
            full(1, W_NUMEL),
            full(MUL_IN, W_NUMEL),
            full(W_NUMEL, PAD_DIM),
            full(SH_DIM, PAD_DIM),
            full(1, PAD_DIM),
        ],
        out_specs=pl.BlockSpec((TC_B, 128), lambda i: (i, 0)),
        out_shape=jax.ShapeDtypeStruct((EH, 128), jnp.float32),
        compiler_params=pltpu.CompilerParams(
            dimension_semantics=("arbitrary",)),
    )(eaT, eaT, xdT, xdT, shT, shT, W1, b1.reshape(1, HID), W2,
      b2.reshape(1, W_NUMEL), Q, M, SHM, E40.reshape(1, PAD_DIM))


def _scatter_body(tp_hbm, src_hbm, zeros_hbm, out_hbm, idx_v, rows_a, rows_b,
                  acc, lsem, ssem):
    c = lax.axis_index("c")
    s = lax.axis_index("s")
    wid = s * NC + c
    base = wid * EPW
    pltpu.sync_copy(zeros_hbm.at[pl.ds(s * NPT, NPT)],
                    acc.at[pl.ds(s * NPT, NPT)])
    pltpu.sync_copy(src_hbm.at[wid], idx_v)
    plsc.subcore_barrier()
    bufs = (rows_a, rows_b)
    load = pltpu.async_copy(tp_hbm.at[pl.ds(base, RROWS)], rows_a, lsem)
    for r in range(RND):
        cur = bufs[r % 2]
        load.wait()
        if r + 1 < RND:
            load = pltpu.async_copy(
                tp_hbm.at[pl.ds(base + (r + 1) * RROWS, RROWS)],
                bufs[(r + 1) % 2], lsem)
        pltpu.async_copy(cur, acc.at[idx_v.at[r]], ssem, add=True).wait()
    plsc.subcore_barrier()
    pltpu.sync_copy(acc.at[pl.ds(s * NPT, NPT)],
                    out_hbm.at[c, pl.ds(s * NPT, NPT)])


@jax.jit
def _sc_scatter(tp, src_r, zeros):
    mesh = plsc.VectorSubcoreMesh(core_axis_name="c", subcore_axis_name="s")
    return pl.kernel(
        _scatter_body,
        out_type=jax.ShapeDtypeStruct((NC, N_PAD, PAD_DIM), jnp.float32),
        mesh=mesh,
        scratch_types=[
            pltpu.VMEM((RND, RROWS), jnp.int32),
            pltpu.VMEM((RROWS, PAD_DIM), jnp.float32),
            pltpu.VMEM((RROWS, PAD_DIM), jnp.float32),
            pltpu.VMEM_SHARED((N_PAD, PAD_DIM), jnp.float32),
            pltpu.SemaphoreType.DMA,
            pltpu.SemaphoreType.DMA,
        ],
        compiler_params=pltpu.CompilerParams(use_tc_tiling_on_sc=False),
    )(tp, src_r, zeros)


CB_B = N_PAD // 8
CNT_A = OUT_DIM
CNT_B = PAD_DIM + OUT_DIM


def _combine_body(p_ref, out_ref):
    s = p_ref[0] + p_ref[1]
    col = lax.broadcasted_iota(jnp.int32, (CB_B, 128), 1)
    cnt_a = jnp.max(jnp.where(col == CNT_A, s, 0.0), axis=1, keepdims=True)
    cnt_b = jnp.max(jnp.where(col == CNT_B, s, 0.0), axis=1, keepdims=True)
    cnt = jnp.where(col < PAD_DIM, jnp.maximum(cnt_a, 1.0),
                    jnp.maximum(cnt_b, 1.0))
    out_ref[...] = s / cnt


@jax.jit
def _tc_combine(partial):
    return pl.pallas_call(
        _combine_body,
        grid=(N_PAD // 2 // CB_B,),
        in_specs=[pl.BlockSpec((NC, CB_B, 128), lambda i: (0, i, 0))],
        out_specs=pl.BlockSpec((CB_B, 128), lambda i: (i, 0)),
        out_shape=jax.ShapeDtypeStruct((N_PAD // 2, 128), jnp.float32),
        compiler_params=pltpu.CompilerParams(
            dimension_semantics=("arbitrary",)),
    )(partial)


def kernel(node_attr, edge_index, edge_attr, edge_sh, W1, b1, W2, b2):
    Q, M, SHM, E40 = _build_consts()
    src = edge_index[0]
    src_perm = jnp.stack([src[:EH], src[EH:]], axis=1).reshape(E)
    src_r = src_perm.reshape(NW, RND, RROWS)
    zeros = jnp.zeros((N_PAD, PAD_DIM), jnp.float32)
    xd = _sc_gather(node_attr, edge_index[1])
    tp2 = _tc_dense(edge_attr.T, xd, edge_sh.T,
                    W1, b1, W2, b2, Q, M, SHM, E40)
    tp = tp2.reshape(E, PAD_DIM)
    partial = _sc_scatter(tp, src_r, zeros)
    out2 = _tc_combine(partial.reshape(NC, N_PAD // 2, 128))
    return out2.reshape(N_PAD, PAD_DIM)[:N, :OUT_DIM]


# speedup vs baseline: 5.5378x; 1.0947x over previous
import numpy as np
import jax
import jax.numpy as jnp
from jax import lax
from jax.experimental import pallas as pl
from jax.experimental.pallas import tpu as pltpu
from jax.experimental.pallas import tpu_sc as plsc

N = 10000
E = 160000
EH = E // 2
MUL_IN = 16
SH_DIM = 4
MUL0_OUT = 16
MUL1_OUT = 8
D_EDGE = 16
HID = 128
W_NUMEL = MUL_IN * MUL0_OUT + MUL_IN * MUL1_OUT
OUT_DIM = MUL0_OUT + 3 * MUL1_OUT
PAD_DIM = 64
NORM = 1.0 / np.sqrt(MUL_IN)

NC = 2
NS = 16
NW = NC * NS
EPW = E // NW
N_PAD = 10240
NPT = N_PAD // NS
RND = 8
RROWS = EPW // RND


def _build_consts():
    q = np.zeros((MUL_IN, W_NUMEL), np.float32)
    for i in range(MUL_IN):
        q[i, i * MUL0_OUT:(i + 1) * MUL0_OUT] = 1.0
        base = MUL_IN * MUL0_OUT
        q[i, base + i * MUL1_OUT: base + (i + 1) * MUL1_OUT] = 1.0
    m = np.zeros((W_NUMEL, PAD_DIM), np.float32)
    for i in range(MUL_IN):
        for o in range(MUL0_OUT):
            m[i * MUL0_OUT + o, o] = NORM
        base = MUL_IN * MUL0_OUT
        for o in range(MUL1_OUT):
            for c in range(3):
                m[base + i * MUL1_OUT + o, MUL0_OUT + o * 3 + c] = NORM
    shm = np.zeros((SH_DIM, PAD_DIM), np.float32)
    shm[0, :MUL0_OUT] = 1.0
    for o in range(MUL1_OUT):
        for c in range(3):
            shm[1 + c, MUL0_OUT + o * 3 + c] = 1.0
    e40 = np.zeros((PAD_DIM,), np.float32)
    e40[OUT_DIM] = 1.0
    return jnp.asarray(q), jnp.asarray(m), jnp.asarray(shm), jnp.asarray(e40)


GC = 250
GCH = EPW // GC


def _gather_body(nodes_hbm, dst_hbm, out_hbm, idx_v, sa, sb, ra, rb,
                 gsem, wsem):
    wid = lax.axis_index("s") * NC + lax.axis_index("c")
    base = wid * EPW
    pltpu.sync_copy(dst_hbm.at[wid], idx_v)
    b16 = (sa, sb)
    b128 = (ra, rb)
    g = pltpu.async_copy(nodes_hbm.at[idx_v.at[0]], sa, gsem)
    w = None
    for c in range(GCH):
        cur16 = b16[c % 2]
        cur128 = b128[c % 2]
        g.wait()
        if w is not None:
            w.wait()
        if c + 1 < GCH:
            g = pltpu.async_copy(nodes_hbm.at[idx_v.at[c + 1]],
                                 b16[(c + 1) % 2], gsem)

        def expand(e, _, cur16=cur16, cur128=cur128):
            cur128[e, pl.ds(0, MUL_IN)] = cur16[e, :]
            return 0

        lax.fori_loop(0, GC, expand, 0)
        w = pltpu.async_copy(cur128, out_hbm.at[pl.ds(base + c * GC, GC)],
                             wsem)
    w.wait()


@jax.jit
def _sc_gather(node_attr, dst_r):
    mesh = plsc.VectorSubcoreMesh(core_axis_name="c", subcore_axis_name="s")
    return pl.kernel(
        _gather_body,
        out_type=jax.ShapeDtypeStruct((E, 128), jnp.float32),
        mesh=mesh,
        scratch_types=[
            pltpu.VMEM((GCH, GC), jnp.int32),
            pltpu.VMEM((GC, MUL_IN), jnp.float32),
            pltpu.VMEM((GC, MUL_IN), jnp.float32),
            pltpu.VMEM((GC, 128), jnp.float32),
            pltpu.VMEM((GC, 128), jnp.float32),
            pltpu.SemaphoreType.DMA,
            pltpu.SemaphoreType.DMA,
        ],
        compiler_params=pltpu.CompilerParams(use_tc_tiling_on_sc=False),
    )(node_attr, dst_r)


TC_B = 3200
GRID = EH // TC_B
def _half(ea_t, xd_t, sh_t, w1, b1, w2, b2, q, m, shm):
    f32 = jnp.float32
    ea = ea_t.T
    xd = xd_t[:, :MUL_IN]
    sh = sh_t.T
    h = jnp.maximum(jnp.dot(ea, w1, preferred_element_type=f32) + b1, 0.0)
    w = jnp.dot(h, w2, preferred_element_type=f32) + b2
    u = w * jnp.dot(xd, q, preferred_element_type=f32)
    return (jnp.dot(u, m, preferred_element_type=f32)
            * jnp.dot(sh, shm, preferred_element_type=f32))


def _dense_body(ea_a, ea_b, xd_a, xd_b, sh_a, sh_b, w1_ref, b1_ref, w2_ref,
                b2_ref, q_ref, m_ref, shm_ref, e40_ref, out_ref):
    w1, b1 = w1_ref[...], b1_ref[...]
    w2, b2 = w2_ref[...], b2_ref[...]
    q, m, shm, e40 = q_ref[...], m_ref[...], shm_ref[...], e40_ref[...]
    tpa = _half(ea_a[...], xd_a[...], sh_a[...], w1, b1, w2, b2, q, m, shm)
    tpb = _half(ea_b[...], xd_b[...], sh_b[...], w1, b1, w2, b2, q, m, shm)
    out_ref[...] = jnp.concatenate([tpa + e40, tpb + e40], axis=1)


@jax.jit
def _tc_dense(eaT, xdT, shT, W1, b1, W2, b2, Q, M, SHM, E40):
    full = lambda r, c: pl.BlockSpec((r, c), lambda i: (0, 0))
    return pl.pallas_call(
        _dense_body,
        grid=(GRID,),
        in_specs=[
            pl.BlockSpec((D_EDGE, TC_B), lambda i: (0, i)),
            pl.BlockSpec((D_EDGE, TC_B), lambda i: (0, i + GRID)),
            pl.BlockSpec((TC_B, 128), lambda i: (i, 0)),
            pl.BlockSpec((TC_B, 128), lambda i: (i + GRID, 0)),
            pl.BlockSpec((SH_DIM, TC_B), lambda i: (0, i)),
            pl.BlockSpec((SH_DIM, TC_B), lambda i: (0, i + GRID)),
            full(D_EDGE, HID),
            full(1, HID),
            full(HID, W_NUMEL),
            full(1, W_NUMEL),
            full(MUL_IN, W_NUMEL),
            full(W_NUMEL, PAD_DIM),
            full(SH_DIM, PAD_DIM),
            full(1, PAD_DIM),
        ],
        out_specs=pl.BlockSpec((TC_B, 128), lambda i: (i, 0)),
        out_shape=jax.ShapeDtypeStruct((EH, 128), jnp.float32),
        compiler_params=pltpu.CompilerParams(
            dimension_semantics=("arbitrary",)),
    )(eaT, eaT, xdT, xdT, shT, shT, W1, b1.reshape(1, HID), W2,
      b2.reshape(1, W_NUMEL), Q, M, SHM, E40.reshape(1, PAD_DIM))


def _scatter_body(tp_hbm, src_hbm, zeros_hbm, out_hbm, idx_v, rows_a, rows_b,
                  acc, lsem, ssem):
    c = lax.axis_index("c")
    s = lax.axis_index("s")
    wid = s * NC + c
    base = wid * EPW
    pltpu.sync_copy(zeros_hbm.at[pl.ds(s * NPT, NPT)],
                    acc.at[pl.ds(s * NPT, NPT)])
    pltpu.sync_copy(src_hbm.at[wid], idx_v)
    plsc.subcore_barrier()
    bufs = (rows_a, rows_b)
    load = pltpu.async_copy(tp_hbm.at[pl.ds(base, RROWS)], rows_a, lsem)
    for r in range(RND):
        cur = bufs[r % 2]
        load.wait()
        if r + 1 < RND:
            load = pltpu.async_copy(
                tp_hbm.at[pl.ds(base + (r + 1) * RROWS, RROWS)],
                bufs[(r + 1) % 2], lsem)
        pltpu.async_copy(cur, acc.at[idx_v.at[r]], ssem, add=True).wait()
    plsc.subcore_barrier()
    pltpu.sync_copy(acc.at[pl.ds(s * NPT, NPT)],
                    out_hbm.at[c, pl.ds(s * NPT, NPT)])


@jax.jit
def _sc_scatter(tp, src_r, zeros):
    mesh = plsc.VectorSubcoreMesh(core_axis_name="c", subcore_axis_name="s")
    return pl.kernel(
        _scatter_body,
        out_type=jax.ShapeDtypeStruct((NC, N_PAD, PAD_DIM), jnp.float32),
        mesh=mesh,
        scratch_types=[
            pltpu.VMEM((RND, RROWS), jnp.int32),
            pltpu.VMEM((RROWS, PAD_DIM), jnp.float32),
            pltpu.VMEM((RROWS, PAD_DIM), jnp.float32),
            pltpu.VMEM_SHARED((N_PAD, PAD_DIM), jnp.float32),
            pltpu.SemaphoreType.DMA,
            pltpu.SemaphoreType.DMA,
        ],
        compiler_params=pltpu.CompilerParams(use_tc_tiling_on_sc=False),
    )(tp, src_r, zeros)


CB_B = N_PAD // 8
CNT_A = OUT_DIM
CNT_B = PAD_DIM + OUT_DIM


def _combine_body(p_ref, out_ref):
    s = p_ref[0] + p_ref[1]
    col = lax.broadcasted_iota(jnp.int32, (CB_B, 128), 1)
    cnt_a = jnp.max(jnp.where(col == CNT_A, s, 0.0), axis=1, keepdims=True)
    cnt_b = jnp.max(jnp.where(col == CNT_B, s, 0.0), axis=1, keepdims=True)
    cnt = jnp.where(col < PAD_DIM, jnp.maximum(cnt_a, 1.0),
                    jnp.maximum(cnt_b, 1.0))
    out_ref[...] = s / cnt


@jax.jit
def _tc_combine(partial):
    return pl.pallas_call(
        _combine_body,
        grid=(N_PAD // 2 // CB_B,),
        in_specs=[pl.BlockSpec((NC, CB_B, 128), lambda i: (0, i, 0))],
        out_specs=pl.BlockSpec((CB_B, 128), lambda i: (i, 0)),
        out_shape=jax.ShapeDtypeStruct((N_PAD // 2, 128), jnp.float32),
        compiler_params=pltpu.CompilerParams(
            dimension_semantics=("arbitrary",)),
    )(partial)


def kernel(node_attr, edge_index, edge_attr, edge_sh, W1, b1, W2, b2):
    Q, M, SHM, E40 = _build_consts()
    src = edge_index[0]
    src_perm = jnp.stack([src[:EH], src[EH:]], axis=1).reshape(E)
    src_r = src_perm.reshape(NW, RND, RROWS)
    zeros = jnp.zeros((N_PAD, PAD_DIM), jnp.float32)
    xd128 = _sc_gather(node_attr, edge_index[1].reshape(NW, GCH, GC))
    tp2 = _tc_dense(edge_attr.T, xd128, edge_sh.T,
                    W1, b1, W2, b2, Q, M, SHM, E40)
    tp = tp2.reshape(E, PAD_DIM)
    partial = _sc_scatter(tp, src_r, zeros)
    out2 = _tc_combine(partial.reshape(NC, N_PAD // 2, 128))
    return out2.reshape(N_PAD, PAD_DIM)[:N, :OUT_DIM]
